# baseline (device time: 244622 ns/iter reference)
import jax
import jax.numpy as jnp
from jax import lax
from jax.experimental import pallas as pl
from jax.experimental.pallas import tpu as pltpu

N_DEV = 4
SQ = 1024
SKV = 1024
D_MODEL = 1024
H_PER = 8
DH = 128
WINDOW = 128
SCALE = 0.08838834764831843
NEG_INF = -1e9
QB = 256
KW = 512


def _body(x_ref, wq_ref, k_hbm, v_hbm, wo_ref, out_ref,
          wq_comm, wo_comm, k_buf, v_buf, stash,
          send_sems, recv_sems, kv_sems, credit_sem, exit_sem):
    my = lax.axis_index("i")
    left = lax.rem(my + N_DEV - 1, N_DEV)
    right = lax.rem(my + 1, N_DEV)

    barrier_sem = pltpu.get_barrier_semaphore()
    for nbr in (left, right):
        pl.semaphore_signal(barrier_sem, inc=1, device_id=(nbr,),
                            device_id_type=pl.DeviceIdType.MESH)
    pl.semaphore_wait(barrier_sem, 2)

    def fetch_kv_start(g):
        ck = pltpu.make_async_copy(
            k_hbm.at[pl.ds(g * H_PER, H_PER)], k_buf, kv_sems.at[0])
        cv = pltpu.make_async_copy(
            v_hbm.at[pl.ds(g * H_PER, H_PER)], v_buf, kv_sems.at[1])
        ck.start()
        cv.start()
        return ck, cv

    def fetch_kv(g):
        ck, cv = fetch_kv_start(g)
        ck.wait()
        cv.wait()

    def block_geom(qb):
        r0 = qb * QB
        w0 = min(max(r0 - WINDOW, 0), SKV - KW)
        qi = lax.broadcasted_iota(jnp.int32, (QB, KW), 0)
        ki = lax.broadcasted_iota(jnp.int32, (QB, KW), 1)
        band = jnp.abs(qi - ki + (r0 - w0)) <= WINDOW
        return r0, w0, band

    def head_ctx(r0, w0, band, h, wq_col):
        qh = jnp.dot(x_ref[r0:r0 + QB], wq_col(h),
                     preferred_element_type=jnp.float32)
        s = lax.dot_general(
            qh, k_buf[h, w0:w0 + KW], (((1,), (1,)), ((), ())),
            preferred_element_type=jnp.float32) * SCALE
        s = jnp.where(band, s, NEG_INF)
        m = jnp.max(s, axis=-1, keepdims=True)
        w = jnp.exp(s - m)
        w = w / jnp.sum(w, axis=-1, keepdims=True)
        return jnp.dot(w, v_buf[h, w0:w0 + KW],
                       preferred_element_type=jnp.float32)

    def term(wq_col, wo_row, first):
        for qb in range(SQ // QB):
            r0, w0, band = block_geom(qb)
            acc = None
            for h in range(H_PER):
                c = head_ctx(r0, w0, band, h, wq_col)
                p = jnp.dot(c, wo_row(h), preferred_element_type=jnp.float32)
                acc = p if acc is None else acc + p
            if first:
                out_ref[r0:r0 + QB] = acc
            else:
                out_ref[r0:r0 + QB] += acc

    def project_stash(wo_row):
        for qb in range(SQ // QB):
            r0 = qb * QB
            acc = None
            for h in range(H_PER):
                p = jnp.dot(stash[r0:r0 + QB, h * DH:(h + 1) * DH], wo_row(h),
                            preferred_element_type=jnp.float32)
                acc = p if acc is None else acc + p
            out_ref[r0:r0 + QB] += acc

    def hop(i, src, dst, target):
        return pltpu.make_async_remote_copy(
            src_ref=src, dst_ref=dst,
            send_sem=send_sems.at[i], recv_sem=recv_sems.at[i],
            device_id=(target,), device_id_type=pl.DeviceIdType.MESH)

    rq0 = hop(0, wq_ref, wq_comm.at[0], right)
    rq1 = hop(1, wq_comm.at[0], wq_comm.at[1], right)
    rq2 = hop(2, wq_comm.at[1], wq_comm.at[0], right)
    ro0 = hop(3, wo_ref, wo_comm.at[0], left)
    ro1 = hop(4, wo_comm.at[0], wo_comm.at[1], left)
    ro2 = hop(5, wo_comm.at[1], wo_comm.at[2], left)

    def wq_slot(s):
        return lambda h: wq_comm[s, :, h * DH:(h + 1) * DH]

    def wo_slot(s):
        return lambda h: wo_comm[s, h * DH:(h + 1) * DH]

    ck, cv = fetch_kv_start(my)
    rq0.start()
    ro0.start()
    ck.wait()
    cv.wait()
    term(lambda h: wq_ref[:, h * DH:(h + 1) * DH],
         lambda h: wo_ref[h * DH:(h + 1) * DH], first=True)

    rq0.wait_recv()
    ro0.wait_recv()
    rq1.start()
    ro1.start()
    fetch_kv(left)
    for qb in range(SQ // QB):
        r0, w0, band = block_geom(qb)
        for h in range(H_PER):
            stash[r0:r0 + QB, h * DH:(h + 1) * DH] = head_ctx(
                r0, w0, band, h, wq_slot(0))
    rq1.wait_send()
    pl.semaphore_signal(credit_sem, inc=1, device_id=(left,),
                        device_id_type=pl.DeviceIdType.MESH)

    rq1.wait_recv()
    ro1.wait_recv()
    pl.semaphore_wait(credit_sem, 1)
    rq2.start()
    ro2.start()
    fetch_kv(lax.rem(my + 2, N_DEV))
    term(wq_slot(1), wo_slot(1), first=False)

    rq2.wait_recv()
    ro2.wait_recv()
    fetch_kv(right)
    term(wq_slot(0), wo_slot(0), first=False)
    project_stash(wo_slot(2))

    for r in (rq0, rq2, ro0, ro1, ro2):
        r.wait_send()

    for nbr in (left, right):
        pl.semaphore_signal(exit_sem, inc=1, device_id=(nbr,),
                            device_id_type=pl.DeviceIdType.MESH)
    pl.semaphore_wait(exit_sem, 2)


def kernel(x, Wq, K_ext, V_ext, Wo):
    my = lax.axis_index("i")
    x2 = x[0]
    k_loc = jnp.transpose(lax.dynamic_index_in_dim(K_ext, my, 0, False),
                          (1, 0, 2))
    v_loc = jnp.transpose(lax.dynamic_index_in_dim(V_ext, my, 0, False),
                          (1, 0, 2))

    out = pl.pallas_call(
        _body,
        out_shape=jax.ShapeDtypeStruct((SQ, D_MODEL), jnp.float32),
        in_specs=[
            pl.BlockSpec(memory_space=pltpu.VMEM),
            pl.BlockSpec(memory_space=pltpu.VMEM),
            pl.BlockSpec(memory_space=pltpu.HBM),
            pl.BlockSpec(memory_space=pltpu.HBM),
            pl.BlockSpec(memory_space=pltpu.VMEM),
        ],
        out_specs=pl.BlockSpec(memory_space=pltpu.VMEM),
        scratch_shapes=[
            pltpu.VMEM((2, D_MODEL, D_MODEL), jnp.float32),
            pltpu.VMEM((3, D_MODEL, D_MODEL), jnp.float32),
            pltpu.VMEM((H_PER, SKV, DH), jnp.float32),
            pltpu.VMEM((H_PER, SKV, DH), jnp.float32),
            pltpu.VMEM((SQ, H_PER * DH), jnp.float32),
            pltpu.SemaphoreType.DMA((6,)),
            pltpu.SemaphoreType.DMA((6,)),
            pltpu.SemaphoreType.DMA((2,)),
            pltpu.SemaphoreType.REGULAR,
            pltpu.SemaphoreType.REGULAR,
        ],
        compiler_params=pltpu.CompilerParams(
            collective_id=0, vmem_limit_bytes=46 * 1024 * 1024
        ),
    )(x2, Wq, k_loc, v_loc, Wo)
    return out[None]


# device time: 234818 ns/iter; 1.0418x vs baseline; 1.0418x over previous
import jax
import jax.numpy as jnp
from jax import lax
from jax.experimental import pallas as pl
from jax.experimental.pallas import tpu as pltpu

N_DEV = 4
SQ = 1024
SKV = 1024
D_MODEL = 1024
H_PER = 8
DH = 128
WINDOW = 128
SCALE = 0.08838834764831843
NEG_INF = -1e9
QB = 256
KW = 512


def _body(x_ref, wq_ref, k_hbm, v_hbm, wo_ref, out_ref,
          wq_comm, wo_comm, k_buf, v_buf, stash,
          send_sems, recv_sems, kv_sems, credit_sem, exit_sem):
    my = lax.axis_index("i")
    left = lax.rem(my + N_DEV - 1, N_DEV)
    right = lax.rem(my + 1, N_DEV)

    barrier_sem = pltpu.get_barrier_semaphore()
    for nbr in (left, right):
        pl.semaphore_signal(barrier_sem, inc=1, device_id=(nbr,),
                            device_id_type=pl.DeviceIdType.MESH)
    pl.semaphore_wait(barrier_sem, 2)

    def fetch_kv_start(g):
        ck = pltpu.make_async_copy(
            k_hbm.at[pl.ds(g * H_PER, H_PER)], k_buf, kv_sems.at[0])
        cv = pltpu.make_async_copy(
            v_hbm.at[pl.ds(g * H_PER, H_PER)], v_buf, kv_sems.at[1])
        ck.start()
        cv.start()
        return ck, cv

    def fetch_kv(g):
        ck, cv = fetch_kv_start(g)
        ck.wait()
        cv.wait()

    band_cache = {}

    def block_geom(qb):
        r0 = qb * QB
        w0 = min(max(r0 - WINDOW, 0), SKV - KW)
        off = r0 - w0
        if off not in band_cache:
            qi = lax.broadcasted_iota(jnp.int32, (QB, KW), 0)
            ki = lax.broadcasted_iota(jnp.int32, (QB, KW), 1)
            band_cache[off] = jnp.where(
                jnp.abs(qi - ki + off) <= WINDOW,
                jnp.float32(0.0), jnp.float32(NEG_INF))
        return r0, w0, band_cache[off]

    def head_ctx(qblk, w0, band, h):
        s = lax.dot_general(
            qblk[:, h * DH:(h + 1) * DH], k_buf[h, w0:w0 + KW],
            (((1,), (1,)), ((), ())),
            preferred_element_type=jnp.float32) * SCALE
        w = jnp.exp(s + band)
        w = w / jnp.sum(w, axis=-1, keepdims=True)
        return jnp.dot(w, v_buf[h, w0:w0 + KW],
                       preferred_element_type=jnp.float32)

    def term(wq_read, wo_row, first):
        for qb in range(SQ // QB):
            r0, w0, band = block_geom(qb)
            qblk = jnp.dot(x_ref[r0:r0 + QB], wq_read(),
                           preferred_element_type=jnp.float32)
            acc = None
            for h in range(H_PER):
                c = head_ctx(qblk, w0, band, h)
                p = jnp.dot(c, wo_row(h), preferred_element_type=jnp.float32)
                acc = p if acc is None else acc + p
            if first:
                out_ref[r0:r0 + QB] = acc
            else:
                out_ref[r0:r0 + QB] += acc

    def project_stash(wo_row):
        for qb in range(SQ // QB):
            r0 = qb * QB
            acc = None
            for h in range(H_PER):
                p = jnp.dot(stash[r0:r0 + QB, h * DH:(h + 1) * DH], wo_row(h),
                            preferred_element_type=jnp.float32)
                acc = p if acc is None else acc + p
            out_ref[r0:r0 + QB] += acc

    def hop(i, src, dst, target):
        return pltpu.make_async_remote_copy(
            src_ref=src, dst_ref=dst,
            send_sem=send_sems.at[i], recv_sem=recv_sems.at[i],
            device_id=(target,), device_id_type=pl.DeviceIdType.MESH)

    rq0 = hop(0, wq_ref, wq_comm.at[0], right)
    rq1 = hop(1, wq_comm.at[0], wq_comm.at[1], right)
    rq2 = hop(2, wq_comm.at[1], wq_comm.at[0], right)
    ro0 = hop(3, wo_ref, wo_comm.at[0], left)
    ro1 = hop(4, wo_comm.at[0], wo_comm.at[1], left)
    ro2 = hop(5, wo_comm.at[1], wo_comm.at[2], left)

    def wq_slot(s):
        return lambda: wq_comm[s]

    def wo_slot(s):
        return lambda h: wo_comm[s, h * DH:(h + 1) * DH]

    ck, cv = fetch_kv_start(my)
    rq0.start()
    ro0.start()
    ck.wait()
    cv.wait()
    term(lambda: wq_ref[...],
         lambda h: wo_ref[h * DH:(h + 1) * DH], first=True)

    rq0.wait_recv()
    ro0.wait_recv()
    rq1.start()
    ro1.start()
    fetch_kv(left)
    for qb in range(SQ // QB):
        r0, w0, band = block_geom(qb)
        qblk = jnp.dot(x_ref[r0:r0 + QB], wq_comm[0],
                       preferred_element_type=jnp.float32)
        for h in range(H_PER):
            stash[r0:r0 + QB, h * DH:(h + 1) * DH] = head_ctx(
                qblk, w0, band, h)
    rq1.wait_send()
    pl.semaphore_signal(credit_sem, inc=1, device_id=(left,),
                        device_id_type=pl.DeviceIdType.MESH)

    rq1.wait_recv()
    ro1.wait_recv()
    pl.semaphore_wait(credit_sem, 1)
    rq2.start()
    ro2.start()
    fetch_kv(lax.rem(my + 2, N_DEV))
    term(wq_slot(1), wo_slot(1), first=False)

    rq2.wait_recv()
    ro2.wait_recv()
    fetch_kv(right)
    term(wq_slot(0), wo_slot(0), first=False)
    project_stash(wo_slot(2))

    for r in (rq0, rq2, ro0, ro1, ro2):
        r.wait_send()

    for nbr in (left, right):
        pl.semaphore_signal(exit_sem, inc=1, device_id=(nbr,),
                            device_id_type=pl.DeviceIdType.MESH)
    pl.semaphore_wait(exit_sem, 2)


def kernel(x, Wq, K_ext, V_ext, Wo):
    my = lax.axis_index("i")
    x2 = x[0]
    k_loc = jnp.transpose(lax.dynamic_index_in_dim(K_ext, my, 0, False),
                          (1, 0, 2))
    v_loc = jnp.transpose(lax.dynamic_index_in_dim(V_ext, my, 0, False),
                          (1, 0, 2))

    out = pl.pallas_call(
        _body,
        out_shape=jax.ShapeDtypeStruct((SQ, D_MODEL), jnp.float32),
        in_specs=[
            pl.BlockSpec(memory_space=pltpu.VMEM),
            pl.BlockSpec(memory_space=pltpu.VMEM),
            pl.BlockSpec(memory_space=pltpu.HBM),
            pl.BlockSpec(memory_space=pltpu.HBM),
            pl.BlockSpec(memory_space=pltpu.VMEM),
        ],
        out_specs=pl.BlockSpec(memory_space=pltpu.VMEM),
        scratch_shapes=[
            pltpu.VMEM((2, D_MODEL, D_MODEL), jnp.float32),
            pltpu.VMEM((3, D_MODEL, D_MODEL), jnp.float32),
            pltpu.VMEM((H_PER, SKV, DH), jnp.float32),
            pltpu.VMEM((H_PER, SKV, DH), jnp.float32),
            pltpu.VMEM((SQ, H_PER * DH), jnp.float32),
            pltpu.SemaphoreType.DMA((6,)),
            pltpu.SemaphoreType.DMA((6,)),
            pltpu.SemaphoreType.DMA((2,)),
            pltpu.SemaphoreType.REGULAR,
            pltpu.SemaphoreType.REGULAR,
        ],
        compiler_params=pltpu.CompilerParams(
            collective_id=0, vmem_limit_bytes=int(47.5 * 1024 * 1024)
        ),
    )(x2, Wq, k_loc, v_loc, Wo)
    return out[None]


# device time: 231147 ns/iter; 1.0583x vs baseline; 1.0159x over previous
import jax
import jax.numpy as jnp
from jax import lax
from jax.experimental import pallas as pl
from jax.experimental.pallas import tpu as pltpu

N_DEV = 4
SQ = 1024
SKV = 1024
D_MODEL = 1024
H_PER = 8
DH = 128
WINDOW = 128
SCALE = 0.08838834764831843
NEG_INF = -1e9
QB = 256
KW = 512


def _body(x_ref, wq_ref, k_hbm, v_hbm, wo_ref, out_ref,
          wq_comm, wo_comm, k_buf, v_buf, stash,
          send_sems, recv_sems, kv_sems, credit_sem, exit_sem):
    my = lax.axis_index("i")
    left = lax.rem(my + N_DEV - 1, N_DEV)
    right = lax.rem(my + 1, N_DEV)

    barrier_sem = pltpu.get_barrier_semaphore()
    for nbr in (left, right):
        pl.semaphore_signal(barrier_sem, inc=1, device_id=(nbr,),
                            device_id_type=pl.DeviceIdType.MESH)
    pl.semaphore_wait(barrier_sem, 2)

    def fetch_kv_start(g):
        ck = pltpu.make_async_copy(
            k_hbm.at[pl.ds(g * H_PER, H_PER)], k_buf, kv_sems.at[0])
        cv = pltpu.make_async_copy(
            v_hbm.at[pl.ds(g * H_PER, H_PER)], v_buf, kv_sems.at[1])
        ck.start()
        cv.start()
        return ck, cv

    def fetch_kv(g):
        ck, cv = fetch_kv_start(g)
        ck.wait()
        cv.wait()

    band_cache = {}

    def block_geom(qb):
        r0 = qb * QB
        w0 = min(max(r0 - WINDOW, 0), SKV - KW)
        off = r0 - w0
        if off not in band_cache:
            qi = lax.broadcasted_iota(jnp.int32, (QB, KW), 0)
            ki = lax.broadcasted_iota(jnp.int32, (QB, KW), 1)
            band_cache[off] = jnp.where(
                jnp.abs(qi - ki + off) <= WINDOW,
                jnp.float32(0.0), jnp.float32(NEG_INF))
        return r0, w0, band_cache[off]

    def head_ctx(qblk, w0, band, h):
        s = lax.dot_general(
            qblk[:, h * DH:(h + 1) * DH], k_buf[h, w0:w0 + KW],
            (((1,), (1,)), ((), ())),
            preferred_element_type=jnp.float32)
        w = jnp.exp(s + band)
        c = jnp.dot(w, v_buf[h, w0:w0 + KW],
                    preferred_element_type=jnp.float32)
        return c / jnp.sum(w, axis=-1, keepdims=True)

    def term(wq_read, wo_row, first):
        for qb in range(SQ // QB):
            r0, w0, band = block_geom(qb)
            qblk = jnp.dot(x_ref[r0:r0 + QB], wq_read(),
                           preferred_element_type=jnp.float32) * SCALE
            acc = None
            for h in range(H_PER):
                c = head_ctx(qblk, w0, band, h)
                p = jnp.dot(c, wo_row(h), preferred_element_type=jnp.float32)
                acc = p if acc is None else acc + p
            if first:
                out_ref[r0:r0 + QB] = acc
            else:
                out_ref[r0:r0 + QB] += acc

    def project_stash(wo_row):
        for qb in range(SQ // QB):
            r0 = qb * QB
            acc = None
            for h in range(H_PER):
                p = jnp.dot(stash[r0:r0 + QB, h * DH:(h + 1) * DH], wo_row(h),
                            preferred_element_type=jnp.float32)
                acc = p if acc is None else acc + p
            out_ref[r0:r0 + QB] += acc

    def hop(i, src, dst, target):
        return pltpu.make_async_remote_copy(
            src_ref=src, dst_ref=dst,
            send_sem=send_sems.at[i], recv_sem=recv_sems.at[i],
            device_id=(target,), device_id_type=pl.DeviceIdType.MESH)

    rq0 = hop(0, wq_ref, wq_comm.at[0], right)
    rq1 = hop(1, wq_comm.at[0], wq_comm.at[1], right)
    rq2 = hop(2, wq_comm.at[1], wq_comm.at[0], right)
    ro0 = hop(3, wo_ref, wo_comm.at[0], left)
    ro1 = hop(4, wo_comm.at[0], wo_comm.at[1], left)
    ro2 = hop(5, wo_comm.at[1], wo_comm.at[2], left)

    def wq_slot(s):
        return lambda: wq_comm[s]

    def wo_slot(s):
        return lambda h: wo_comm[s, h * DH:(h + 1) * DH]

    ck, cv = fetch_kv_start(my)
    rq0.start()
    ro0.start()
    ck.wait()
    cv.wait()
    term(lambda: wq_ref[...],
         lambda h: wo_ref[h * DH:(h + 1) * DH], first=True)

    rq0.wait_recv()
    ro0.wait_recv()
    rq1.start()
    ro1.start()
    fetch_kv(left)
    for qb in range(SQ // QB):
        r0, w0, band = block_geom(qb)
        qblk = jnp.dot(x_ref[r0:r0 + QB], wq_comm[0],
                       preferred_element_type=jnp.float32) * SCALE
        for h in range(H_PER):
            stash[r0:r0 + QB, h * DH:(h + 1) * DH] = head_ctx(
                qblk, w0, band, h)
    rq1.wait_send()
    pl.semaphore_signal(credit_sem, inc=1, device_id=(left,),
                        device_id_type=pl.DeviceIdType.MESH)

    rq1.wait_recv()
    ro1.wait_recv()
    pl.semaphore_wait(credit_sem, 1)
    rq2.start()
    ro2.start()
    fetch_kv(lax.rem(my + 2, N_DEV))
    term(wq_slot(1), wo_slot(1), first=False)

    rq2.wait_recv()
    ro2.wait_recv()
    fetch_kv(right)
    term(wq_slot(0), wo_slot(0), first=False)
    project_stash(wo_slot(2))

    for r in (rq0, rq2, ro0, ro1, ro2):
        r.wait_send()

    for nbr in (left, right):
        pl.semaphore_signal(exit_sem, inc=1, device_id=(nbr,),
                            device_id_type=pl.DeviceIdType.MESH)
    pl.semaphore_wait(exit_sem, 2)


def kernel(x, Wq, K_ext, V_ext, Wo):
    my = lax.axis_index("i")
    x2 = x[0]
    k_loc = jnp.transpose(lax.dynamic_index_in_dim(K_ext, my, 0, False),
                          (1, 0, 2))
    v_loc = jnp.transpose(lax.dynamic_index_in_dim(V_ext, my, 0, False),
                          (1, 0, 2))

    out = pl.pallas_call(
        _body,
        out_shape=jax.ShapeDtypeStruct((SQ, D_MODEL), jnp.float32),
        in_specs=[
            pl.BlockSpec(memory_space=pltpu.VMEM),
            pl.BlockSpec(memory_space=pltpu.VMEM),
            pl.BlockSpec(memory_space=pltpu.HBM),
            pl.BlockSpec(memory_space=pltpu.HBM),
            pl.BlockSpec(memory_space=pltpu.VMEM),
        ],
        out_specs=pl.BlockSpec(memory_space=pltpu.VMEM),
        scratch_shapes=[
            pltpu.VMEM((2, D_MODEL, D_MODEL), jnp.float32),
            pltpu.VMEM((3, D_MODEL, D_MODEL), jnp.float32),
            pltpu.VMEM((H_PER, SKV, DH), jnp.float32),
            pltpu.VMEM((H_PER, SKV, DH), jnp.float32),
            pltpu.VMEM((SQ, H_PER * DH), jnp.float32),
            pltpu.SemaphoreType.DMA((6,)),
            pltpu.SemaphoreType.DMA((6,)),
            pltpu.SemaphoreType.DMA((2,)),
            pltpu.SemaphoreType.REGULAR,
            pltpu.SemaphoreType.REGULAR,
        ],
        compiler_params=pltpu.CompilerParams(
            collective_id=0, vmem_limit_bytes=int(47.5 * 1024 * 1024)
        ),
    )(x2, Wq, k_loc, v_loc, Wo)
    return out[None]


# device time: 200944 ns/iter; 1.2174x vs baseline; 1.1503x over previous
import jax
import jax.numpy as jnp
from jax import lax
from jax.experimental import pallas as pl
from jax.experimental.pallas import tpu as pltpu

N_DEV = 4
SQ = 1024
SKV = 1024
D_MODEL = 1024
H_PER = 8
DH = 128
WINDOW = 128
SCALE = 0.08838834764831843
NEG_INF = -1e9
QB = 256
KW = 512


def _body(x_ref, wq_ref, k_hbm, v_hbm, wo_ref, out_ref,
          wq_comm, wo_comm, k_buf, v_buf, stash,
          send_sems, recv_sems, kv_sems, credit_sem, exit_sem):
    my = lax.axis_index("i")
    left = lax.rem(my + N_DEV - 1, N_DEV)
    right = lax.rem(my + 1, N_DEV)

    barrier_sem = pltpu.get_barrier_semaphore()
    for nbr in (left, right):
        pl.semaphore_signal(barrier_sem, inc=1, device_id=(nbr,),
                            device_id_type=pl.DeviceIdType.MESH)
    pl.semaphore_wait(barrier_sem, 2)

    def fetch_kv_start(g):
        copies = []
        for h in range(H_PER):
            copies.append(pltpu.make_async_copy(
                k_hbm.at[:, g * H_PER + h, :], k_buf.at[h], kv_sems.at[0]))
            copies.append(pltpu.make_async_copy(
                v_hbm.at[:, g * H_PER + h, :], v_buf.at[h], kv_sems.at[1]))
        for c in copies:
            c.start()
        return copies

    def fetch_kv(g):
        for c in fetch_kv_start(g):
            c.wait()

    band_cache = {}

    def block_geom(qb):
        r0 = qb * QB
        w0 = min(max(r0 - WINDOW, 0), SKV - KW)
        off = r0 - w0
        if off not in band_cache:
            qi = lax.broadcasted_iota(jnp.int32, (QB, KW), 0)
            ki = lax.broadcasted_iota(jnp.int32, (QB, KW), 1)
            band_cache[off] = jnp.where(
                jnp.abs(qi - ki + off) <= WINDOW,
                jnp.float32(0.0), jnp.float32(NEG_INF))
        return r0, w0, band_cache[off]

    def head_ctx(qblk, w0, band, h):
        s = lax.dot_general(
            qblk[:, h * DH:(h + 1) * DH], k_buf[h, w0:w0 + KW],
            (((1,), (1,)), ((), ())),
            preferred_element_type=jnp.float32)
        w = jnp.exp(s + band)
        c = jnp.dot(w, v_buf[h, w0:w0 + KW],
                    preferred_element_type=jnp.float32)
        return c / jnp.sum(w, axis=-1, keepdims=True)

    def term(wq_read, wo_row, first):
        for qb in range(SQ // QB):
            r0, w0, band = block_geom(qb)
            qblk = jnp.dot(x_ref[r0:r0 + QB], wq_read(),
                           preferred_element_type=jnp.float32) * SCALE
            acc = None
            for h in range(H_PER):
                c = head_ctx(qblk, w0, band, h)
                p = jnp.dot(c, wo_row(h), preferred_element_type=jnp.float32)
                acc = p if acc is None else acc + p
            if first:
                out_ref[r0:r0 + QB] = acc
            else:
                out_ref[r0:r0 + QB] += acc

    def project_stash(wo_row):
        for qb in range(SQ // QB):
            r0 = qb * QB
            acc = None
            for h in range(H_PER):
                p = jnp.dot(stash[r0:r0 + QB, h * DH:(h + 1) * DH], wo_row(h),
                            preferred_element_type=jnp.float32)
                acc = p if acc is None else acc + p
            out_ref[r0:r0 + QB] += acc

    def hop(i, src, dst, target):
        return pltpu.make_async_remote_copy(
            src_ref=src, dst_ref=dst,
            send_sem=send_sems.at[i], recv_sem=recv_sems.at[i],
            device_id=(target,), device_id_type=pl.DeviceIdType.MESH)

    rq0 = hop(0, wq_ref, wq_comm.at[0], right)
    rq1 = hop(1, wq_comm.at[0], wq_comm.at[1], right)
    rq2 = hop(2, wq_comm.at[1], wq_comm.at[0], right)
    ro0 = hop(3, wo_ref, wo_comm.at[0], left)
    ro1 = hop(4, wo_comm.at[0], wo_comm.at[1], left)
    ro2 = hop(5, wo_comm.at[1], wo_comm.at[2], left)

    def wq_slot(s):
        return lambda: wq_comm[s]

    def wo_slot(s):
        return lambda h: wo_comm[s, h * DH:(h + 1) * DH]

    copies0 = fetch_kv_start(my)
    rq0.start()
    ro0.start()
    for c in copies0:
        c.wait()
    term(lambda: wq_ref[...],
         lambda h: wo_ref[h * DH:(h + 1) * DH], first=True)

    rq0.wait_recv()
    ro0.wait_recv()
    rq1.start()
    ro1.start()
    fetch_kv(left)
    for qb in range(SQ // QB):
        r0, w0, band = block_geom(qb)
        qblk = jnp.dot(x_ref[r0:r0 + QB], wq_comm[0],
                       preferred_element_type=jnp.float32) * SCALE
        for h in range(H_PER):
            stash[r0:r0 + QB, h * DH:(h + 1) * DH] = head_ctx(
                qblk, w0, band, h)
    rq1.wait_send()
    pl.semaphore_signal(credit_sem, inc=1, device_id=(left,),
                        device_id_type=pl.DeviceIdType.MESH)

    rq1.wait_recv()
    ro1.wait_recv()
    pl.semaphore_wait(credit_sem, 1)
    rq2.start()
    ro2.start()
    fetch_kv(lax.rem(my + 2, N_DEV))
    term(wq_slot(1), wo_slot(1), first=False)

    rq2.wait_recv()
    ro2.wait_recv()
    fetch_kv(right)
    term(wq_slot(0), wo_slot(0), first=False)
    project_stash(wo_slot(2))

    for r in (rq0, rq2, ro0, ro1, ro2):
        r.wait_send()

    for nbr in (left, right):
        pl.semaphore_signal(exit_sem, inc=1, device_id=(nbr,),
                            device_id_type=pl.DeviceIdType.MESH)
    pl.semaphore_wait(exit_sem, 2)


def kernel(x, Wq, K_ext, V_ext, Wo):
    my = lax.axis_index("i")
    x2 = x[0]
    k_loc = lax.dynamic_index_in_dim(K_ext, my, 0, False)
    v_loc = lax.dynamic_index_in_dim(V_ext, my, 0, False)

    out = pl.pallas_call(
        _body,
        out_shape=jax.ShapeDtypeStruct((SQ, D_MODEL), jnp.float32),
        in_specs=[
            pl.BlockSpec(memory_space=pltpu.VMEM),
            pl.BlockSpec(memory_space=pltpu.VMEM),
            pl.BlockSpec(memory_space=pltpu.HBM),
            pl.BlockSpec(memory_space=pltpu.HBM),
            pl.BlockSpec(memory_space=pltpu.VMEM),
        ],
        out_specs=pl.BlockSpec(memory_space=pltpu.VMEM),
        scratch_shapes=[
            pltpu.VMEM((2, D_MODEL, D_MODEL), jnp.float32),
            pltpu.VMEM((3, D_MODEL, D_MODEL), jnp.float32),
            pltpu.VMEM((H_PER, SKV, DH), jnp.float32),
            pltpu.VMEM((H_PER, SKV, DH), jnp.float32),
            pltpu.VMEM((SQ, H_PER * DH), jnp.float32),
            pltpu.SemaphoreType.DMA((6,)),
            pltpu.SemaphoreType.DMA((6,)),
            pltpu.SemaphoreType.DMA((2,)),
            pltpu.SemaphoreType.REGULAR,
            pltpu.SemaphoreType.REGULAR,
        ],
        compiler_params=pltpu.CompilerParams(
            collective_id=0, vmem_limit_bytes=int(47.5 * 1024 * 1024)
        ),
    )(x2, Wq, k_loc, v_loc, Wo)
    return out[None]


# device time: 178687 ns/iter; 1.3690x vs baseline; 1.1246x over previous
import jax
import jax.numpy as jnp
from jax import lax
from jax.experimental import pallas as pl
from jax.experimental.pallas import tpu as pltpu

N_DEV = 4
SQ = 1024
SKV = 1024
D_MODEL = 1024
H_PER = 8
DH = 128
WINDOW = 128
SCALE = 0.08838834764831843
NEG_INF = -1e9
QB = 256
KW = 512


def _body(x_ref, wq_ref, k_hbm, v_hbm, wo_ref, out_ref,
          wq_comm, wo_comm, k_buf, v_buf, stash,
          send_sems, recv_sems, kv_sems, credit_sem, exit_sem):
    my = lax.axis_index("i")
    left = lax.rem(my + N_DEV - 1, N_DEV)
    right = lax.rem(my + 1, N_DEV)

    barrier_sem = pltpu.get_barrier_semaphore()
    for nbr in (left, right):
        pl.semaphore_signal(barrier_sem, inc=1, device_id=(nbr,),
                            device_id_type=pl.DeviceIdType.MESH)
    pl.semaphore_wait(barrier_sem, 2)

    def fetch_kv_start(g):
        copies = []
        for h in range(H_PER):
            copies.append(pltpu.make_async_copy(
                k_hbm.at[my, :, g * H_PER + h, :], k_buf.at[h],
                kv_sems.at[0]))
            copies.append(pltpu.make_async_copy(
                v_hbm.at[my, :, g * H_PER + h, :], v_buf.at[h],
                kv_sems.at[1]))
        for c in copies:
            c.start()
        return copies

    def fetch_kv(g):
        for c in fetch_kv_start(g):
            c.wait()

    band_cache = {}

    def block_geom(qb):
        r0 = qb * QB
        w0 = min(max(r0 - WINDOW, 0), SKV - KW)
        off = r0 - w0
        if off not in band_cache:
            qi = lax.broadcasted_iota(jnp.int32, (QB, KW), 0)
            ki = lax.broadcasted_iota(jnp.int32, (QB, KW), 1)
            band_cache[off] = jnp.where(
                jnp.abs(qi - ki + off) <= WINDOW,
                jnp.float32(0.0), jnp.float32(NEG_INF))
        return r0, w0, band_cache[off]

    def head_ctx(qblk, w0, band, h):
        s = lax.dot_general(
            qblk[:, h * DH:(h + 1) * DH], k_buf[h, w0:w0 + KW],
            (((1,), (1,)), ((), ())),
            preferred_element_type=jnp.float32)
        w = jnp.exp(s + band)
        c = jnp.dot(w, v_buf[h, w0:w0 + KW],
                    preferred_element_type=jnp.float32)
        return c / jnp.sum(w, axis=-1, keepdims=True)

    def term(wq_read, wo_row, first):
        for qb in range(SQ // QB):
            r0, w0, band = block_geom(qb)
            qblk = jnp.dot(x_ref[r0:r0 + QB], wq_read(),
                           preferred_element_type=jnp.float32) * SCALE
            acc = None
            for h in range(H_PER):
                c = head_ctx(qblk, w0, band, h)
                p = jnp.dot(c, wo_row(h), preferred_element_type=jnp.float32)
                acc = p if acc is None else acc + p
            if first:
                out_ref[r0:r0 + QB] = acc
            else:
                out_ref[r0:r0 + QB] += acc

    def project_stash(wo_row):
        for qb in range(SQ // QB):
            r0 = qb * QB
            acc = None
            for h in range(H_PER):
                p = jnp.dot(stash[r0:r0 + QB, h * DH:(h + 1) * DH], wo_row(h),
                            preferred_element_type=jnp.float32)
                acc = p if acc is None else acc + p
            out_ref[r0:r0 + QB] += acc

    def hop(i, src, dst, target):
        return pltpu.make_async_remote_copy(
            src_ref=src, dst_ref=dst,
            send_sem=send_sems.at[i], recv_sem=recv_sems.at[i],
            device_id=(target,), device_id_type=pl.DeviceIdType.MESH)

    rq0 = hop(0, wq_ref, wq_comm.at[0], right)
    rq1 = hop(1, wq_comm.at[0], wq_comm.at[1], right)
    rq2 = hop(2, wq_comm.at[1], wq_comm.at[0], right)
    ro0 = hop(3, wo_ref, wo_comm.at[0], left)
    ro1 = hop(4, wo_comm.at[0], wo_comm.at[1], left)
    ro2 = hop(5, wo_comm.at[1], wo_comm.at[2], left)

    def wq_slot(s):
        return lambda: wq_comm[s]

    def wo_slot(s):
        return lambda h: wo_comm[s, h * DH:(h + 1) * DH]

    copies0 = fetch_kv_start(my)
    rq0.start()
    ro0.start()
    for c in copies0:
        c.wait()
    term(lambda: wq_ref[...],
         lambda h: wo_ref[h * DH:(h + 1) * DH], first=True)

    rq0.wait_recv()
    ro0.wait_recv()
    rq1.start()
    ro1.start()
    fetch_kv(left)
    for qb in range(SQ // QB):
        r0, w0, band = block_geom(qb)
        qblk = jnp.dot(x_ref[r0:r0 + QB], wq_comm[0],
                       preferred_element_type=jnp.float32) * SCALE
        for h in range(H_PER):
            stash[r0:r0 + QB, h * DH:(h + 1) * DH] = head_ctx(
                qblk, w0, band, h)
    rq1.wait_send()
    pl.semaphore_signal(credit_sem, inc=1, device_id=(left,),
                        device_id_type=pl.DeviceIdType.MESH)

    rq1.wait_recv()
    ro1.wait_recv()
    pl.semaphore_wait(credit_sem, 1)
    rq2.start()
    ro2.start()
    fetch_kv(lax.rem(my + 2, N_DEV))
    term(wq_slot(1), wo_slot(1), first=False)

    rq2.wait_recv()
    ro2.wait_recv()
    fetch_kv(right)
    term(wq_slot(0), wo_slot(0), first=False)
    project_stash(wo_slot(2))

    for r in (rq0, rq2, ro0, ro1, ro2):
        r.wait_send()

    for nbr in (left, right):
        pl.semaphore_signal(exit_sem, inc=1, device_id=(nbr,),
                            device_id_type=pl.DeviceIdType.MESH)
    pl.semaphore_wait(exit_sem, 2)


def kernel(x, Wq, K_ext, V_ext, Wo):
    x2 = x[0]

    out = pl.pallas_call(
        _body,
        out_shape=jax.ShapeDtypeStruct((SQ, D_MODEL), jnp.float32),
        in_specs=[
            pl.BlockSpec(memory_space=pltpu.VMEM),
            pl.BlockSpec(memory_space=pltpu.VMEM),
            pl.BlockSpec(memory_space=pltpu.HBM),
            pl.BlockSpec(memory_space=pltpu.HBM),
            pl.BlockSpec(memory_space=pltpu.VMEM),
        ],
        out_specs=pl.BlockSpec(memory_space=pltpu.VMEM),
        scratch_shapes=[
            pltpu.VMEM((2, D_MODEL, D_MODEL), jnp.float32),
            pltpu.VMEM((3, D_MODEL, D_MODEL), jnp.float32),
            pltpu.VMEM((H_PER, SKV, DH), jnp.float32),
            pltpu.VMEM((H_PER, SKV, DH), jnp.float32),
            pltpu.VMEM((SQ, H_PER * DH), jnp.float32),
            pltpu.SemaphoreType.DMA((6,)),
            pltpu.SemaphoreType.DMA((6,)),
            pltpu.SemaphoreType.DMA((2,)),
            pltpu.SemaphoreType.REGULAR,
            pltpu.SemaphoreType.REGULAR,
        ],
        compiler_params=pltpu.CompilerParams(
            collective_id=0, vmem_limit_bytes=int(47.5 * 1024 * 1024)
        ),
    )(x2, Wq, K_ext, V_ext, Wo)
    return out[None]
